# baseline (speedup 1.0000x reference)
import functools

import numpy as np
import jax
import jax.numpy as jnp
from jax import lax
from jax.experimental import pallas as pl

D = 1024
K_HI = 72
K_LO = 128
R = 512


@functools.lru_cache(maxsize=None)
def _trig_tables():
    j = np.arange(D, dtype=np.float64)
    f = np.power(10000.0, -2.0 * np.floor(j / 2.0) / D)
    p = np.where(j % 2 == 1, np.pi / 2.0, 0.0)
    h = np.arange(K_HI, dtype=np.float64)[:, None]
    SH = np.sin(h * 128.0 * f[None, :])
    CH = np.cos(h * 128.0 * f[None, :])
    l = np.arange(K_LO, dtype=np.float64)[:, None]
    SL = np.sin(l * f[None, :] + p[None, :])
    CL = np.cos(l * f[None, :] + p[None, :])
    TH = np.concatenate([SH, CH], axis=1)
    TL = np.concatenate([CL, SL], axis=1)
    return jnp.asarray(TH, jnp.bfloat16), jnp.asarray(TL, jnp.bfloat16)


def _tc_body(idx_ref, th_ref, tl_ref, out_ref):
    idxc = idx_ref[0]
    hi = idxc >> 7
    lo = idxc & 127
    oh_hi = (hi == lax.broadcasted_iota(jnp.int32, (R, K_HI), 1)).astype(jnp.bfloat16)
    oh_lo = (lo == lax.broadcasted_iota(jnp.int32, (R, K_LO), 1)).astype(jnp.bfloat16)
    g_hi = jnp.dot(oh_hi, th_ref[...], preferred_element_type=jnp.float32)
    g_lo = jnp.dot(oh_lo, tl_ref[...], preferred_element_type=jnp.float32)
    res = g_hi[:, :D] * g_lo[:, :D] + g_hi[:, D:] * g_lo[:, D:]
    out_ref[0] = jnp.where(idxc != 0, res, 0.0)


def kernel(input_batch, table):
    shape = input_batch.shape
    idx = input_batch.reshape(-1).astype(jnp.int32)
    B = idx.shape[0]
    nblocks = B // R
    th, tl = _trig_tables()
    idx3 = idx.reshape(nblocks, R, 1)
    out = pl.pallas_call(
        _tc_body,
        grid=(nblocks,),
        in_specs=[
            pl.BlockSpec((1, R, 1), lambda i: (i, 0, 0)),
            pl.BlockSpec((K_HI, 2 * D), lambda i: (0, 0)),
            pl.BlockSpec((K_LO, 2 * D), lambda i: (0, 0)),
        ],
        out_specs=pl.BlockSpec((1, R, D), lambda i: (i, 0, 0)),
        out_shape=jax.ShapeDtypeStruct((nblocks, R, D), jnp.float32),
    )(idx3, th, tl)
    return out.reshape(*shape, D)
